# baseline (speedup 1.0000x reference)
import functools

import jax
import jax.numpy as jnp
from jax import lax
from jax.experimental import pallas as pl
from jax.experimental.pallas import tpu as pltpu
from jax.experimental.pallas import tpu_sc as plsc

N = 10000
E = 320000
D = 128
OUT = 128
T = 10
L = 40
SPLIT = 7
TL = T * L
TRSEG = SPLIT * L

NP = 10240
NC = 2
NS = 16
NW = NC * NS
RPT = NP // NS
B = 128
EP = 327680
EPC = EP // NC
EPS = EPC // NS
NBLK_E = EPS // B
CHB = NBLK_E // 2
TLP = 512
SRT = TLP // NS
BN = 80
RPW = NP // NW
NBLK_N = RPW // BN

_mesh = plsc.VectorSubcoreMesh(core_axis_name="c", subcore_axis_name="s")


def _wid(c, s):
    return c * NS + s


@functools.partial(
    pl.kernel,
    out_type=jax.ShapeDtypeStruct((NW * NP,), jnp.float32),
    mesh=_mesh,
    scratch_types=[
        pltpu.VMEM((B,), jnp.int32),
        pltpu.VMEM((B,), jnp.int32),
        pltpu.VMEM((NP,), jnp.float32),
        pltpu.SemaphoreType.DMA,
        pltpu.SemaphoreType.DMA,
    ],
    compiler_params=pltpu.CompilerParams(needs_layout_passes=False),
)
def _deg_kernel(dst_hbm, out_hbm, d0, d1, hist, sd0, sd1):
    c = lax.axis_index("c")
    s = lax.axis_index("s")
    zeros16 = jnp.zeros((16,), jnp.float32)

    def zbody(i, _):
        hist[pl.ds(i * 16, 16)] = zeros16
        return 0

    lax.fori_loop(0, NP // 16, zbody, 0)
    base0 = _wid(c, s) * EPS
    ones16 = jnp.ones((16,), jnp.float32)

    def ld(i, buf, sem):
        pltpu.async_copy(dst_hbm.at[pl.ds(base0 + i * B, B)], buf, sem)

    def use(i, buf, sem):
        pltpu.make_async_copy(dst_hbm.at[pl.ds(base0 + i * B, B)],
                              buf, sem).wait()
        for j in range(B // 16):
            idx = buf[pl.ds(j * 16, 16)]
            plsc.addupdate_scatter(hist, [idx], ones16)

    ld(0, d0, sd0)
    ld(1, d1, sd1)

    def body(k, _):
        i0 = 2 * k
        use(i0, d0, sd0)
        ld(i0 + 2, d0, sd0)
        use(i0 + 1, d1, sd1)
        ld(i0 + 3, d1, sd1)
        return 0

    lax.fori_loop(0, (NBLK_E - 2) // 2, body, 0)
    use(NBLK_E - 2, d0, sd0)
    use(NBLK_E - 1, d1, sd1)
    pltpu.sync_copy(hist, out_hbm.at[pl.ds(_wid(c, s) * NP, NP)])


@functools.partial(
    pl.kernel,
    out_type=jax.ShapeDtypeStruct((NC * NP, D), jnp.float32),
    mesh=_mesh,
    scratch_types=[
        pltpu.VMEM((B,), jnp.int32),
        pltpu.VMEM((B,), jnp.int32),
        pltpu.VMEM((B,), jnp.int32),
        pltpu.VMEM((B,), jnp.int32),
        pltpu.VMEM((B, D), jnp.float32),
        pltpu.VMEM((B, D), jnp.float32),
        pltpu.VMEM_SHARED((NP, D), jnp.float32),
        pltpu.SemaphoreType.DMA,
        pltpu.SemaphoreType.DMA,
        pltpu.SemaphoreType.DMA,
        pltpu.SemaphoreType.DMA,
        pltpu.SemaphoreType.DMA,
        pltpu.SemaphoreType.DMA,
    ],
)
def _hop_kernel(g_hbm, src_hbm, dst_hbm, z_hbm, out_hbm,
                si0, si1, di0, di1, rows0, rows1, acc,
                ss0, ss1, sd0, sd1, sg0, sg1):
    c = lax.axis_index("c")
    s = lax.axis_index("s")
    base = _wid(c, s) * EPS
    pltpu.sync_copy(z_hbm, acc.at[pl.ds(s * RPT, RPT)])
    plsc.subcore_barrier()

    def ld(hbm, i, buf, sem):
        pltpu.async_copy(hbm.at[pl.ds(base + i * B, B)], buf, sem)

    def wt(hbm, i, buf, sem):
        pltpu.make_async_copy(hbm.at[pl.ds(base + i * B, B)], buf, sem).wait()

    ld(src_hbm, 0, si0, ss0)
    ld(dst_hbm, 0, di0, sd0)
    ld(src_hbm, 1, si1, ss1)
    ld(dst_hbm, 1, di1, sd1)
    wt(src_hbm, 0, si0, ss0)
    pltpu.async_copy(g_hbm.at[si0], rows0, sg0)
    wt(src_hbm, 1, si1, ss1)
    pltpu.async_copy(g_hbm.at[si1], rows1, sg1)

    def half(i, si, di, rows, ss, sd, sg, last):
        pltpu.make_async_copy(g_hbm.at[si], rows, sg).wait()
        if not last:
            ld(src_hbm, i + 2, si, ss)
        wt(dst_hbm, i, di, sd)
        pltpu.sync_copy(rows, acc.at[di], add=True)
        if not last:
            ld(dst_hbm, i + 2, di, sd)
            wt(src_hbm, i + 2, si, ss)
            pltpu.async_copy(g_hbm.at[si], rows, sg)

    def body(k, _):
        i0 = 2 * k
        half(i0, si0, di0, rows0, ss0, sd0, sg0, False)
        half(i0 + 1, si1, di1, rows1, ss1, sd1, sg1, False)
        return 0

    lax.fori_loop(0, (NBLK_E - 2) // 2, body, 0)
    half(NBLK_E - 2, si0, di0, rows0, ss0, sd0, sg0, True)
    half(NBLK_E - 1, si1, di1, rows1, ss1, sd1, sg1, True)
    plsc.subcore_barrier()
    pltpu.sync_copy(acc.at[pl.ds(s * RPT, RPT)],
                    out_hbm.at[pl.ds(c * NP + s * RPT, RPT)])


@functools.partial(
    pl.kernel,
    out_type=(
        jax.ShapeDtypeStruct((NC * TLP, D), jnp.float32),
        jax.ShapeDtypeStruct((NC * TLP, D), jnp.float32),
        jax.ShapeDtypeStruct((NC * TLP, D), jnp.float32),
    ),
    mesh=_mesh,
    scratch_types=[
        pltpu.VMEM((BN,), jnp.int32),
        pltpu.VMEM((BN, D), jnp.float32),
        pltpu.VMEM((BN, D), jnp.float32),
        pltpu.VMEM((BN, D), jnp.float32),
        pltpu.VMEM_SHARED((TLP, D), jnp.float32),
        pltpu.VMEM_SHARED((TLP, D), jnp.float32),
        pltpu.VMEM_SHARED((TLP, D), jnp.float32),
    ],
)
def _stats_kernel(h_hbm, q_hbm, ones_hbm, seg_hbm, z_hbm,
                  outh_hbm, outq_hbm, outc_hbm,
                  seg_v, rows_h, rows_q, ones_v, acc_h, acc_q, acc_c):
    c = lax.axis_index("c")
    s = lax.axis_index("s")
    pltpu.sync_copy(z_hbm, acc_h.at[pl.ds(s * SRT, SRT)])
    pltpu.sync_copy(z_hbm, acc_q.at[pl.ds(s * SRT, SRT)])
    pltpu.sync_copy(z_hbm, acc_c.at[pl.ds(s * SRT, SRT)])
    pltpu.sync_copy(ones_hbm, ones_v)
    plsc.subcore_barrier()
    base0 = _wid(c, s) * RPW

    def body(i, _):
        base = base0 + i * BN
        pltpu.sync_copy(seg_hbm.at[pl.ds(base, BN)], seg_v)
        pltpu.sync_copy(h_hbm.at[pl.ds(base, BN)], rows_h)
        pltpu.sync_copy(q_hbm.at[pl.ds(base, BN)], rows_q)
        pltpu.sync_copy(rows_h, acc_h.at[seg_v], add=True)
        pltpu.sync_copy(rows_q, acc_q.at[seg_v], add=True)
        pltpu.sync_copy(ones_v, acc_c.at[seg_v], add=True)
        return 0

    lax.fori_loop(0, NBLK_N, body, 0)
    plsc.subcore_barrier()
    sl_s = pl.ds(s * SRT, SRT)
    sl_o = pl.ds(c * TLP + s * SRT, SRT)
    pltpu.sync_copy(acc_h.at[sl_s], outh_hbm.at[sl_o])
    pltpu.sync_copy(acc_q.at[sl_s], outq_hbm.at[sl_o])
    pltpu.sync_copy(acc_c.at[sl_s], outc_hbm.at[sl_o])


@functools.partial(
    pl.kernel,
    out_type=jax.ShapeDtypeStruct((NP, D), jnp.float32),
    mesh=_mesh,
    scratch_types=[
        pltpu.VMEM((BN,), jnp.int32),
        pltpu.VMEM((BN, D), jnp.float32),
        pltpu.SemaphoreType.DMA,
    ],
)
def _gather_kernel(tbl_hbm, seg_hbm, out_hbm, seg_v, rows_v, sem):
    c = lax.axis_index("c")
    s = lax.axis_index("s")
    base0 = _wid(c, s) * RPW

    def body(i, _):
        base = base0 + i * BN
        pltpu.sync_copy(seg_hbm.at[pl.ds(base, BN)], seg_v)
        pltpu.async_copy(tbl_hbm.at[seg_v], rows_v, sem).wait()
        pltpu.sync_copy(rows_v, out_hbm.at[pl.ds(base, BN)])
        return 0

    lax.fori_loop(0, NBLK_N, body, 0)


_BR = 256


def _final_body(h_ref, tmg_ref, af_ref, v_ref, bt_ref, out_ref):
    af = af_ref[...]
    hf = af * h_ref[...] + (1.0 - af) * tmg_ref[...]
    out_ref[...] = jax.lax.dot_general(
        hf, v_ref[...], (((1,), (0,)), ((), ())),
        preferred_element_type=jnp.float32,
        precision=jax.lax.Precision.HIGHEST,
    ) + bt_ref[...]


def _final_tc(h, tmg, af, V, btil):
    grid = (NP // _BR,)
    return pl.pallas_call(
        _final_body,
        grid=grid,
        in_specs=[
            pl.BlockSpec((_BR, D), lambda i: (i, 0)),
            pl.BlockSpec((_BR, D), lambda i: (i, 0)),
            pl.BlockSpec((_BR, 1), lambda i: (i, 0)),
            pl.BlockSpec((D, OUT), lambda i: (0, 0)),
            pl.BlockSpec((1, OUT), lambda i: (0, 0)),
        ],
        out_specs=pl.BlockSpec((_BR, OUT), lambda i: (i, 0)),
        out_shape=jax.ShapeDtypeStruct((NP, OUT), jnp.float32),
    )(h, tmg, af, V, btil)


def kernel(x, edge_index, labels, times, W, b):
    f32 = jnp.float32
    pad = jnp.full((EP - E,), NP - 1, jnp.int32)
    src = jnp.concatenate([edge_index[0], pad])
    dst = jnp.concatenate([edge_index[1], pad])
    zrow = jnp.zeros((RPT, D), f32)
    zst = jnp.zeros((SRT, D), f32)
    ones_bd = jnp.ones((BN, D), f32)

    degp = _deg_kernel(dst)
    deg = jnp.sum(degp.reshape(NW, NP), axis=0)[:, None]
    norm = jnp.power(jnp.maximum(deg, 1.0), -0.5)

    xpad = jnp.zeros((NP, D), f32).at[:N].set(x)
    g = xpad * norm
    p = _hop_kernel(g, src, dst, zrow)
    s1 = p[:NP] + p[NP:]
    g = s1 * (norm * norm)
    p = _hop_kernel(g, src, dst, zrow)
    h = (p[:NP] + p[NP:]) * norm

    seg = times * L + labels
    segp = jnp.full((NP,), TLP - 1, jnp.int32).at[:N].set(seg)
    q = h * h
    oh, oq, oc = _stats_kernel(h, q, ones_bd, segp, zst)
    ssum = (oh[:TLP] + oh[TLP:])[:TL]
    ssq = (oq[:TLP] + oq[TLP:])[:TL]
    cnt = (oc[:TLP] + oc[TLP:])[:TL, 0]

    tr_cnt = cnt[:TRSEG].reshape(SPLIT, L)
    tr_sum = ssum[:TRSEG].reshape(SPLIT, L, D)
    tr_ssq = ssq[:TRSEG].reshape(SPLIT, L, D)
    test_cnt = jnp.sum(cnt[TRSEG:])
    test_sum = jnp.sum(ssum[TRSEG:], axis=0)
    test_ssq = jnp.sum(ssq[TRSEG:], axis=0)
    test_mean = test_sum / jnp.maximum(test_cnt, 1.0)
    test_var = (jnp.sum(test_ssq) - 2.0 * jnp.dot(test_mean, test_sum)
                + test_cnt * jnp.dot(test_mean, test_mean)
                ) / jnp.maximum(test_cnt - 1.0, 1.0)
    time_cnt = jnp.sum(tr_cnt, axis=1)
    ttm = jnp.sum(tr_sum, axis=1) / jnp.maximum(time_cnt, 1.0)[:, None]
    tm = tr_sum / jnp.maximum(tr_cnt, 1.0)[:, :, None]
    msq = jnp.sum(tr_cnt * jnp.sum((tm - ttm[:, None, :]) ** 2, axis=2), axis=1)
    rsq = jnp.sum(jnp.sum(tr_ssq, axis=2)
                  - 2.0 * jnp.sum(tm * tr_sum, axis=2)
                  + tr_cnt * jnp.sum(tm * tm, axis=2), axis=1)
    denom = jnp.maximum(time_cnt - 1.0, 1.0)
    alpha_sq = (test_var - msq / denom) / jnp.maximum(rsq / denom, 1e-6)
    alpha7 = jnp.where(alpha_sq > 0, jnp.sqrt(jnp.maximum(alpha_sq, 0.0)), 0.0)

    tot_cnt = jnp.sum(cnt)
    mu = (jnp.sum(tr_sum, axis=(0, 1)) + test_sum) / tot_cnt
    a2 = (alpha7 ** 2)[:, None, None]
    blend_ssq = a2 * tr_ssq + (1.0 - a2) * (tr_sum ** 2) \
        / jnp.maximum(tr_cnt, 1.0)[:, :, None]
    col_ssq = jnp.sum(blend_ssq, axis=(0, 1)) + test_ssq
    sigma = jnp.sqrt((col_ssq - tot_cnt * mu * mu) / (tot_cnt - 1.0))
    V = (W / sigma[None, :]).T
    btil = (b - (mu / sigma) @ W.T)[None, :]

    tm_tbl = jnp.zeros((TLP, D), f32).at[:TRSEG].set(tm.reshape(TRSEG, D))
    tmg = _gather_kernel(tm_tbl, segp)
    alpha10 = jnp.concatenate([alpha7, jnp.ones((T - SPLIT,), f32)])
    af = jnp.ones((NP, 1), f32).at[:N, 0].set(alpha10[times])

    out = _final_tc(h, tmg, af, V, btil)
    return out[:N]


# speedup vs baseline: 3.1556x; 3.1556x over previous
import functools

import jax
import jax.numpy as jnp
from jax import lax
from jax.experimental import pallas as pl
from jax.experimental.pallas import tpu as pltpu
from jax.experimental.pallas import tpu_sc as plsc

N = 10000
E = 320000
D = 128
OUT = 128
T = 10
L = 40
SPLIT = 7
TL = T * L
TRSEG = SPLIT * L

NP = 10240
NC = 2
NS = 16
NW = NC * NS
RPT = NP // NS
B = 128
EP = 327680
EPC = EP // NC
EPS = EPC // NS
NBLK_E = EPS // B
CHB = NBLK_E // 2
TLP = 512
SRT = TLP // NS
BN = 80
RPW = NP // NW
NBLK_N = RPW // BN

_mesh = plsc.VectorSubcoreMesh(core_axis_name="c", subcore_axis_name="s")


def _wid(c, s):
    return c * NS + s


@functools.partial(
    pl.kernel,
    out_type=jax.ShapeDtypeStruct((NW * NP,), jnp.float32),
    mesh=_mesh,
    scratch_types=[
        pltpu.VMEM((B,), jnp.int32),
        pltpu.VMEM((B,), jnp.int32),
        pltpu.VMEM((NP,), jnp.float32),
        pltpu.SemaphoreType.DMA,
        pltpu.SemaphoreType.DMA,
    ],
    compiler_params=pltpu.CompilerParams(needs_layout_passes=False),
)
def _deg_kernel(dst_hbm, out_hbm, d0, d1, hist, sd0, sd1):
    c = lax.axis_index("c")
    s = lax.axis_index("s")
    zeros16 = jnp.zeros((16,), jnp.float32)

    def zbody(i, _):
        hist[pl.ds(i * 16, 16)] = zeros16
        return 0

    lax.fori_loop(0, NP // 16, zbody, 0)
    base0 = _wid(c, s) * EPS
    ones16 = jnp.ones((16,), jnp.float32)

    def ld(i, buf, sem):
        pltpu.async_copy(dst_hbm.at[pl.ds(base0 + i * B, B)], buf, sem)

    def use(i, buf, sem):
        pltpu.make_async_copy(dst_hbm.at[pl.ds(base0 + i * B, B)],
                              buf, sem).wait()
        for j in range(B // 16):
            idx = buf[pl.ds(j * 16, 16)]
            plsc.addupdate_scatter(hist, [idx], ones16)

    ld(0, d0, sd0)
    ld(1, d1, sd1)

    def body(k, _):
        i0 = 2 * k
        use(i0, d0, sd0)
        ld(i0 + 2, d0, sd0)
        use(i0 + 1, d1, sd1)
        ld(i0 + 3, d1, sd1)
        return 0

    lax.fori_loop(0, (NBLK_E - 2) // 2, body, 0)
    use(NBLK_E - 2, d0, sd0)
    use(NBLK_E - 1, d1, sd1)
    pltpu.sync_copy(hist, out_hbm.at[pl.ds(_wid(c, s) * NP, NP)])


@functools.partial(
    pl.kernel,
    out_type=jax.ShapeDtypeStruct((NC * NP, D), jnp.float32),
    mesh=_mesh,
    scratch_types=[
        pltpu.VMEM((B,), jnp.int32),
        pltpu.VMEM((B,), jnp.int32),
        pltpu.VMEM((B,), jnp.int32),
        pltpu.VMEM((B,), jnp.int32),
        pltpu.VMEM((B, D), jnp.float32),
        pltpu.VMEM((B, D), jnp.float32),
        pltpu.VMEM_SHARED((NP, D), jnp.float32),
        pltpu.SemaphoreType.DMA,
        pltpu.SemaphoreType.DMA,
        pltpu.SemaphoreType.DMA,
        pltpu.SemaphoreType.DMA,
        pltpu.SemaphoreType.DMA,
        pltpu.SemaphoreType.DMA,
    ],
)
def _hop_kernel(g_hbm, src_hbm, dst_hbm, z_hbm, out_hbm,
                si0, si1, di0, di1, rows0, rows1, acc,
                ss0, ss1, sd0, sd1, sg0, sg1):
    c = lax.axis_index("c")
    s = lax.axis_index("s")
    base = _wid(c, s) * EPS
    pltpu.sync_copy(z_hbm, acc.at[pl.ds(s * RPT, RPT)])
    plsc.subcore_barrier()

    def ld(hbm, i, buf, sem):
        pltpu.async_copy(hbm.at[pl.ds(base + i * B, B)], buf, sem)

    def wt(hbm, i, buf, sem):
        pltpu.make_async_copy(hbm.at[pl.ds(base + i * B, B)], buf, sem).wait()

    ld(src_hbm, 0, si0, ss0)
    ld(dst_hbm, 0, di0, sd0)
    ld(src_hbm, 1, si1, ss1)
    ld(dst_hbm, 1, di1, sd1)
    wt(src_hbm, 0, si0, ss0)
    pltpu.async_copy(g_hbm.at[si0], rows0, sg0)
    wt(src_hbm, 1, si1, ss1)
    pltpu.async_copy(g_hbm.at[si1], rows1, sg1)

    def half(i, si, di, rows, ss, sd, sg, last):
        pltpu.make_async_copy(g_hbm.at[si], rows, sg).wait()
        if not last:
            ld(src_hbm, i + 2, si, ss)
        wt(dst_hbm, i, di, sd)
        pltpu.sync_copy(rows, acc.at[di], add=True)
        if not last:
            ld(dst_hbm, i + 2, di, sd)
            wt(src_hbm, i + 2, si, ss)
            pltpu.async_copy(g_hbm.at[si], rows, sg)

    def body(k, _):
        i0 = 2 * k
        half(i0, si0, di0, rows0, ss0, sd0, sg0, False)
        half(i0 + 1, si1, di1, rows1, ss1, sd1, sg1, False)
        return 0

    lax.fori_loop(0, (NBLK_E - 2) // 2, body, 0)
    half(NBLK_E - 2, si0, di0, rows0, ss0, sd0, sg0, True)
    half(NBLK_E - 1, si1, di1, rows1, ss1, sd1, sg1, True)
    plsc.subcore_barrier()
    pltpu.sync_copy(acc.at[pl.ds(s * RPT, RPT)],
                    out_hbm.at[pl.ds(c * NP + s * RPT, RPT)])


@functools.partial(
    pl.kernel,
    out_type=(
        jax.ShapeDtypeStruct((NC * TLP, D), jnp.float32),
        jax.ShapeDtypeStruct((NC * TLP, D), jnp.float32),
        jax.ShapeDtypeStruct((NC * TLP, D), jnp.float32),
    ),
    mesh=_mesh,
    scratch_types=[
        pltpu.VMEM((BN,), jnp.int32),
        pltpu.VMEM((BN, D), jnp.float32),
        pltpu.VMEM((BN, D), jnp.float32),
        pltpu.VMEM((BN, D), jnp.float32),
        pltpu.VMEM_SHARED((TLP, D), jnp.float32),
        pltpu.VMEM_SHARED((TLP, D), jnp.float32),
        pltpu.VMEM_SHARED((TLP, D), jnp.float32),
    ],
)
def _stats_kernel(h_hbm, q_hbm, ones_hbm, seg_hbm, z_hbm,
                  outh_hbm, outq_hbm, outc_hbm,
                  seg_v, rows_h, rows_q, ones_v, acc_h, acc_q, acc_c):
    c = lax.axis_index("c")
    s = lax.axis_index("s")
    pltpu.sync_copy(z_hbm, acc_h.at[pl.ds(s * SRT, SRT)])
    pltpu.sync_copy(z_hbm, acc_q.at[pl.ds(s * SRT, SRT)])
    pltpu.sync_copy(z_hbm, acc_c.at[pl.ds(s * SRT, SRT)])
    pltpu.sync_copy(ones_hbm, ones_v)
    plsc.subcore_barrier()
    base0 = _wid(c, s) * RPW

    def body(i, _):
        base = base0 + i * BN
        pltpu.sync_copy(seg_hbm.at[pl.ds(base, BN)], seg_v)
        pltpu.sync_copy(h_hbm.at[pl.ds(base, BN)], rows_h)
        pltpu.sync_copy(q_hbm.at[pl.ds(base, BN)], rows_q)
        pltpu.sync_copy(rows_h, acc_h.at[seg_v], add=True)
        pltpu.sync_copy(rows_q, acc_q.at[seg_v], add=True)
        pltpu.sync_copy(ones_v, acc_c.at[seg_v], add=True)
        return 0

    lax.fori_loop(0, NBLK_N, body, 0)
    plsc.subcore_barrier()
    sl_s = pl.ds(s * SRT, SRT)
    sl_o = pl.ds(c * TLP + s * SRT, SRT)
    pltpu.sync_copy(acc_h.at[sl_s], outh_hbm.at[sl_o])
    pltpu.sync_copy(acc_q.at[sl_s], outq_hbm.at[sl_o])
    pltpu.sync_copy(acc_c.at[sl_s], outc_hbm.at[sl_o])


@functools.partial(
    pl.kernel,
    out_type=jax.ShapeDtypeStruct((NP, D), jnp.float32),
    mesh=_mesh,
    scratch_types=[
        pltpu.VMEM((BN,), jnp.int32),
        pltpu.VMEM((BN, D), jnp.float32),
        pltpu.SemaphoreType.DMA,
    ],
)
def _gather_kernel(tbl_hbm, seg_hbm, out_hbm, seg_v, rows_v, sem):
    c = lax.axis_index("c")
    s = lax.axis_index("s")
    base0 = _wid(c, s) * RPW

    def body(i, _):
        base = base0 + i * BN
        pltpu.sync_copy(seg_hbm.at[pl.ds(base, BN)], seg_v)
        pltpu.async_copy(tbl_hbm.at[seg_v], rows_v, sem).wait()
        pltpu.sync_copy(rows_v, out_hbm.at[pl.ds(base, BN)])
        return 0

    lax.fori_loop(0, NBLK_N, body, 0)


_BR = 256


def _final_body(h_ref, tmg_ref, af_ref, v_ref, bt_ref, out_ref):
    af = af_ref[...]
    hf = af * h_ref[...] + (1.0 - af) * tmg_ref[...]
    out_ref[...] = jax.lax.dot_general(
        hf, v_ref[...], (((1,), (0,)), ((), ())),
        preferred_element_type=jnp.float32,
        precision=jax.lax.Precision.HIGHEST,
    ) + bt_ref[...]


def _final_tc(h, tmg, af, V, btil):
    grid = (NP // _BR,)
    return pl.pallas_call(
        _final_body,
        grid=grid,
        in_specs=[
            pl.BlockSpec((_BR, D), lambda i: (i, 0)),
            pl.BlockSpec((_BR, D), lambda i: (i, 0)),
            pl.BlockSpec((_BR, 1), lambda i: (i, 0)),
            pl.BlockSpec((D, OUT), lambda i: (0, 0)),
            pl.BlockSpec((1, OUT), lambda i: (0, 0)),
        ],
        out_specs=pl.BlockSpec((_BR, OUT), lambda i: (i, 0)),
        out_shape=jax.ShapeDtypeStruct((NP, OUT), jnp.float32),
    )(h, tmg, af, V, btil)


def kernel(x, edge_index, labels, times, W, b):
    f32 = jnp.float32
    pad = N + (jnp.arange(EP - E, dtype=jnp.int32) % (NP - N))
    src = jnp.concatenate([edge_index[0], pad])
    dst = jnp.concatenate([edge_index[1], pad])
    zrow = jnp.zeros((RPT, D), f32)
    zst = jnp.zeros((SRT, D), f32)
    ones_bd = jnp.ones((BN, D), f32)

    degp = _deg_kernel(dst)
    deg = jnp.sum(degp.reshape(NW, NP), axis=0)[:, None]
    norm = jnp.power(jnp.maximum(deg, 1.0), -0.5)

    xpad = jnp.zeros((NP, D), f32).at[:N].set(x)
    g = xpad * norm
    p = _hop_kernel(g, src, dst, zrow)
    s1 = p[:NP] + p[NP:]
    g = s1 * (norm * norm)
    p = _hop_kernel(g, src, dst, zrow)
    h = (p[:NP] + p[NP:]) * norm

    seg = times * L + labels
    segp = jnp.full((NP,), TLP - 1, jnp.int32).at[:N].set(seg)
    q = h * h
    oh, oq, oc = _stats_kernel(h, q, ones_bd, segp, zst)
    ssum = (oh[:TLP] + oh[TLP:])[:TL]
    ssq = (oq[:TLP] + oq[TLP:])[:TL]
    cnt = (oc[:TLP] + oc[TLP:])[:TL, 0]

    tr_cnt = cnt[:TRSEG].reshape(SPLIT, L)
    tr_sum = ssum[:TRSEG].reshape(SPLIT, L, D)
    tr_ssq = ssq[:TRSEG].reshape(SPLIT, L, D)
    test_cnt = jnp.sum(cnt[TRSEG:])
    test_sum = jnp.sum(ssum[TRSEG:], axis=0)
    test_ssq = jnp.sum(ssq[TRSEG:], axis=0)
    test_mean = test_sum / jnp.maximum(test_cnt, 1.0)
    test_var = (jnp.sum(test_ssq) - 2.0 * jnp.dot(test_mean, test_sum)
                + test_cnt * jnp.dot(test_mean, test_mean)
                ) / jnp.maximum(test_cnt - 1.0, 1.0)
    time_cnt = jnp.sum(tr_cnt, axis=1)
    ttm = jnp.sum(tr_sum, axis=1) / jnp.maximum(time_cnt, 1.0)[:, None]
    tm = tr_sum / jnp.maximum(tr_cnt, 1.0)[:, :, None]
    msq = jnp.sum(tr_cnt * jnp.sum((tm - ttm[:, None, :]) ** 2, axis=2), axis=1)
    rsq = jnp.sum(jnp.sum(tr_ssq, axis=2)
                  - 2.0 * jnp.sum(tm * tr_sum, axis=2)
                  + tr_cnt * jnp.sum(tm * tm, axis=2), axis=1)
    denom = jnp.maximum(time_cnt - 1.0, 1.0)
    alpha_sq = (test_var - msq / denom) / jnp.maximum(rsq / denom, 1e-6)
    alpha7 = jnp.where(alpha_sq > 0, jnp.sqrt(jnp.maximum(alpha_sq, 0.0)), 0.0)

    tot_cnt = jnp.sum(cnt)
    mu = (jnp.sum(tr_sum, axis=(0, 1)) + test_sum) / tot_cnt
    a2 = (alpha7 ** 2)[:, None, None]
    blend_ssq = a2 * tr_ssq + (1.0 - a2) * (tr_sum ** 2) \
        / jnp.maximum(tr_cnt, 1.0)[:, :, None]
    col_ssq = jnp.sum(blend_ssq, axis=(0, 1)) + test_ssq
    sigma = jnp.sqrt((col_ssq - tot_cnt * mu * mu) / (tot_cnt - 1.0))
    V = (W / sigma[None, :]).T
    btil = (b - (mu / sigma) @ W.T)[None, :]

    tm_tbl = jnp.zeros((TLP, D), f32).at[:TRSEG].set(tm.reshape(TRSEG, D))
    tmg = _gather_kernel(tm_tbl, segp)
    alpha10 = jnp.concatenate([alpha7, jnp.ones((T - SPLIT,), f32)])
    af = jnp.ones((NP, 1), f32).at[:N, 0].set(alpha10[times])

    out = _final_tc(h, tmg, af, V, btil)
    return out[:N]
